# baseline (speedup 1.0000x reference)
import jax
import jax.numpy as jnp
from jax import lax
from jax.experimental import pallas as pl
from jax.experimental.pallas import tpu as pltpu
from jax.experimental.pallas import tpu_sc as plsc

N = 10000
E = 320000
FEAT = 128
HEADS = 8
HID = 16
DM = HEADS * HID

NC = 2
NS = 16
NW = NC * NS

E_TOT = E + N
EPB = 64
BLOCKS_PER_W = -(-E_TOT // (NW * EPB))
B_C0 = 180
B_C1 = 144
E_PAD = NW * EPB * BLOCKS_PER_W
ROWS_PER_SUB = N // NS

_EPS = 1e-16


def _tc1_body(x_ref, w_ref, as_ref, ad_ref, perm_ref, h_ref, als_ref,
              ald_ref):
  h = jnp.dot(x_ref[...], w_ref[...], preferred_element_type=jnp.float32)
  h_ref[...] = jnp.dot(h, perm_ref[...],
                       preferred_element_type=jnp.float32).astype(jnp.bfloat16)
  als_ref[...] = jnp.dot(h, as_ref[...], preferred_element_type=jnp.float32)
  ald_ref[...] = jnp.dot(h, ad_ref[...], preferred_element_type=jnp.float32)


def _tc1(x, w, a_s, a_d, perm):
  grid = 10
  blk = N // grid
  return pl.pallas_call(
      _tc1_body,
      grid=(grid,),
      in_specs=[
          pl.BlockSpec((blk, FEAT), lambda i: (i, 0)),
          pl.BlockSpec((FEAT, DM), lambda i: (0, 0)),
          pl.BlockSpec((DM, 16), lambda i: (0, 0)),
          pl.BlockSpec((DM, 16), lambda i: (0, 0)),
          pl.BlockSpec((DM, DM), lambda i: (0, 0)),
      ],
      out_specs=[
          pl.BlockSpec((blk, DM), lambda i: (i, 0)),
          pl.BlockSpec((blk, 16), lambda i: (i, 0)),
          pl.BlockSpec((blk, 16), lambda i: (i, 0)),
      ],
      out_shape=[
          jax.ShapeDtypeStruct((N, DM), jnp.bfloat16),
          jax.ShapeDtypeStruct((N, 16), jnp.float32),
          jax.ShapeDtypeStruct((N, 16), jnp.float32),
      ],
  )(x, w, a_s, a_d, perm)


def _tc2_body(u_ref, s_ref, exp_ref, b_ref, w_ref, as_ref, ad_ref, perm_ref,
              h2_ref, als_ref, ald_ref):
  u = u_ref[0] + u_ref[1]
  st = s_ref[0] + s_ref[1]
  r = 1.0 / (st + _EPS)
  rex = jnp.dot(r, exp_ref[...], preferred_element_type=jnp.float32)
  h1 = jnp.maximum(u * rex + b_ref[...], 0.0)
  h2 = jnp.dot(h1, w_ref[...], preferred_element_type=jnp.float32)
  h2_ref[...] = jnp.dot(h2, perm_ref[...],
                        preferred_element_type=jnp.float32).astype(jnp.bfloat16)
  als_ref[...] = jnp.dot(h2, as_ref[...], preferred_element_type=jnp.float32)
  ald_ref[...] = jnp.dot(h2, ad_ref[...], preferred_element_type=jnp.float32)


def _tc2(u, s, expand, b1, w2, a_s, a_d, perm):
  grid = 10
  blk = N // grid
  return pl.pallas_call(
      _tc2_body,
      grid=(grid,),
      in_specs=[
          pl.BlockSpec((2, blk, DM), lambda i: (0, i, 0)),
          pl.BlockSpec((2, blk, 16), lambda i: (0, i, 0)),
          pl.BlockSpec((16, DM), lambda i: (0, 0)),
          pl.BlockSpec((1, DM), lambda i: (0, 0)),
          pl.BlockSpec((DM, DM), lambda i: (0, 0)),
          pl.BlockSpec((DM, 16), lambda i: (0, 0)),
          pl.BlockSpec((DM, 16), lambda i: (0, 0)),
          pl.BlockSpec((DM, DM), lambda i: (0, 0)),
      ],
      out_specs=[
          pl.BlockSpec((blk, DM), lambda i: (i, 0)),
          pl.BlockSpec((blk, 16), lambda i: (i, 0)),
          pl.BlockSpec((blk, 16), lambda i: (i, 0)),
      ],
      out_shape=[
          jax.ShapeDtypeStruct((N, DM), jnp.bfloat16),
          jax.ShapeDtypeStruct((N, 16), jnp.float32),
          jax.ShapeDtypeStruct((N, 16), jnp.float32),
      ],
  )(u, s, expand, b1, w2, a_s, a_d, perm)


def _tc3_body(u_ref, s_ref, exp_ref, red_ref, b_ref, wp_ref, bp_ref, o_ref):
  u = u_ref[0] + u_ref[1]
  st = s_ref[0] + s_ref[1]
  r = 1.0 / (st + _EPS)
  rex = jnp.dot(r, exp_ref[...], preferred_element_type=jnp.float32)
  o = jnp.dot(u * rex, red_ref[...], preferred_element_type=jnp.float32)
  z = jnp.dot(o + b_ref[...], wp_ref[...], preferred_element_type=jnp.float32)
  o_ref[...] = jax.nn.sigmoid(z + bp_ref[...])


def _tc3(u, s, expand, red, b2, wp, bp):
  grid = 10
  blk = N // grid
  return pl.pallas_call(
      _tc3_body,
      grid=(grid,),
      in_specs=[
          pl.BlockSpec((2, blk, DM), lambda i: (0, i, 0)),
          pl.BlockSpec((2, blk, 16), lambda i: (0, i, 0)),
          pl.BlockSpec((16, DM), lambda i: (0, 0)),
          pl.BlockSpec((DM, 16), lambda i: (0, 0)),
          pl.BlockSpec((1, 16), lambda i: (0, 0)),
          pl.BlockSpec((16, 8), lambda i: (0, 0)),
          pl.BlockSpec((1, 8), lambda i: (0, 0)),
      ],
      out_specs=pl.BlockSpec((blk, 8), lambda i: (i, 0)),
      out_shape=jax.ShapeDtypeStruct((N, 8), jnp.float32),
  )(u, s, expand, red, b2, wp, bp)


def _sc_edge_body(src_hbm, dst_hbm, h_hbm, als_hbm, ald_hbm, z128_hbm,
                  z16_hbm, u_out, s_out,
                  src_v, dst_v, h_buf, out_buf, als_buf, ald_buf, p_buf,
                  u_shared, s_shared, sem_g0, sem_g1, sem_g2,
                  sem_i0, sem_i1, sem_i2, sem_i3, sem_i4, sem_i5, sem_sc):
  c = lax.axis_index("c")
  s = lax.axis_index("s")
  wid = s * NC + c

  nbase = s * ROWS_PER_SUB
  pltpu.sync_copy(z128_hbm, u_shared.at[pl.ds(nbase, ROWS_PER_SUB)])
  pltpu.sync_copy(z16_hbm, s_shared.at[pl.ds(nbase, ROWS_PER_SUB)])

  row0 = s * (B_C0 + B_C1) + c * B_C0
  nblocks = jnp.where(c == 0, B_C0, B_C1)
  sem_g = (sem_g0, sem_g1, sem_g2)
  sem_i = (sem_i0, sem_i1, sem_i2, sem_i3, sem_i4, sem_i5)

  def idx_copies(j, islot):
    return (
        pltpu.make_async_copy(src_hbm.at[row0 + j], src_v.at[islot],
                              sem_i[islot]),
        pltpu.make_async_copy(dst_hbm.at[row0 + j], dst_v.at[islot],
                              sem_i[islot]),
    )

  def gathers(hslot, islot):
    return (
        pltpu.make_async_copy(h_hbm.at[src_v.at[islot]], h_buf.at[hslot],
                              sem_g[hslot]),
        pltpu.make_async_copy(als_hbm.at[src_v.at[islot]], als_buf.at[hslot],
                              sem_g[hslot]),
        pltpu.make_async_copy(ald_hbm.at[dst_v.at[islot]], ald_buf.at[hslot],
                              sem_g[hslot]),
    )

  def scatters(oslot, hslot, islot):
    return (
        pltpu.make_async_copy(out_buf.at[oslot], u_shared.at[dst_v.at[islot]],
                              sem_sc),
        pltpu.make_async_copy(p_buf.at[hslot], s_shared.at[dst_v.at[islot]],
                              sem_sc),
    )

  def start(cps, add=False):
    for cp in cps:
      cp.start(add=add)

  def wait(cps):
    for cp in cps:
      cp.wait()

  def compute(j, hslot, oslot):
    gid0 = (row0 + j) * EPB

    @plsc.parallel_loop(0, EPB, unroll=6)
    def _(e):
      a = als_buf[hslot, e, :]
      d = ald_buf[hslot, e, :]
      t = a + d
      t = jnp.maximum(t, 0.2 * t)
      p = jnp.exp(t)
      valid = ((gid0 + e) < E_TOT).astype(jnp.float32)
      p = p * valid
      p_buf[hslot, e, :] = p
      for g4 in range(4):
        v = h_buf[hslot, e, pl.ds(g4 * 32, 32)]
        ha, hb = plsc.unpack(v, format=plsc.PackFormat.INTERLEAVED)
        out_buf[oslot, e, pl.ds(g4 * 32, HID)] = ha * p[2 * g4]
        out_buf[oslot, e, pl.ds(g4 * 32 + HID, HID)] = hb * p[2 * g4 + 1]

  for jj in range(4):
    start(idx_copies(jj, jj))
  wait(idx_copies(0, 0))
  wait(idx_copies(1, 1))
  start(gathers(0, 0))
  start(gathers(1, 1))

  def six_blocks(g, _):
    j0 = 6 * g
    for k in range(6):
      j = j0 + k
      hs = k % 3

      wait(gathers(hs, k))

      @pl.when(j >= 1)
      def _():
        wait(scatters((k + 1) % 2, (k + 2) % 3, (k + 5) % 6))

      compute(j, hs, k % 2)
      start(scatters(k % 2, hs, k), add=True)

      @pl.when(j + 2 < nblocks)
      def _():
        wait(idx_copies(j + 2, (k + 2) % 6))
        start(gathers((k + 2) % 3, (k + 2) % 6))

      @pl.when(j + 4 < nblocks)
      def _():
        start(idx_copies(j + 4, (k + 4) % 6))

    return 0

  lax.fori_loop(0, nblocks // 6, six_blocks, 0)
  wait(scatters(1, 2, 5))

  plsc.subcore_barrier()

  pltpu.sync_copy(u_shared.at[pl.ds(nbase, ROWS_PER_SUB)],
                  u_out.at[c, pl.ds(nbase, ROWS_PER_SUB)])
  pltpu.sync_copy(s_shared.at[pl.ds(nbase, ROWS_PER_SUB)],
                  s_out.at[c, pl.ds(nbase, ROWS_PER_SUB)])


def _sc_edge(src2d, dst2d, h, als, ald, z128, z16):
  mesh = plsc.VectorSubcoreMesh(core_axis_name="c", subcore_axis_name="s")
  fn = pl.kernel(
      _sc_edge_body,
      out_type=[
          jax.ShapeDtypeStruct((NC, N, DM), jnp.float32),
          jax.ShapeDtypeStruct((NC, N, 16), jnp.float32),
      ],
      mesh=mesh,
      compiler_params=pltpu.CompilerParams(use_tc_tiling_on_sc=False,
                                           needs_layout_passes=False),
      scratch_types=[
          pltpu.VMEM((6, EPB), jnp.int32),
          pltpu.VMEM((6, EPB), jnp.int32),
          pltpu.VMEM((3, EPB, DM), jnp.bfloat16),
          pltpu.VMEM((2, EPB, DM), jnp.float32),
          pltpu.VMEM((3, EPB, 16), jnp.float32),
          pltpu.VMEM((3, EPB, 16), jnp.float32),
          pltpu.VMEM((3, EPB, 16), jnp.float32),
          pltpu.VMEM_SHARED((N, DM), jnp.float32),
          pltpu.VMEM_SHARED((N, 16), jnp.float32),
      ] + [pltpu.SemaphoreType.DMA] * 10,
  )
  return fn(src2d, dst2d, h, als, ald, z128, z16)


def _head_proj(a):
  flat = a.reshape(DM)
  rows = jnp.arange(DM)
  return jnp.zeros((DM, 16), jnp.float32).at[rows, rows // HID].set(flat)


def _wp_pad(wp):
  return jnp.concatenate([wp, jnp.zeros((HID, 7), jnp.float32)], axis=1)


@jax.jit
def kernel(edge_index, x, W1, a_src1, a_dst1, b1, W2, a_src2, a_dst2, b2,
           Wp, bp):
  loops = jnp.arange(N, dtype=jnp.int32)
  pad = jnp.zeros((E_PAD - E_TOT,), jnp.int32)
  src2d = jnp.concatenate(
      [edge_index[0].astype(jnp.int32), loops, pad]).reshape(-1, EPB)
  dst2d = jnp.concatenate(
      [edge_index[1].astype(jnp.int32), loops, pad]).reshape(-1, EPB)

  as1 = _head_proj(a_src1)
  ad1 = _head_proj(a_dst1)
  as2 = _head_proj(a_src2)
  ad2 = _head_proj(a_dst2)

  expand = (jnp.arange(DM)[None, :] // HID ==
            jnp.arange(16)[:, None]).astype(jnp.float32)
  red = ((jnp.arange(DM)[:, None] % HID ==
          jnp.arange(16)[None, :]).astype(jnp.float32) / HEADS)

  z128 = jnp.zeros((ROWS_PER_SUB, DM), jnp.float32)
  z16 = jnp.zeros((ROWS_PER_SUB, 16), jnp.float32)

  gg = jnp.arange(4)[:, None, None]
  hh = jnp.arange(2)[None, :, None]
  ii = jnp.arange(16)[None, None, :]
  in_cols = (32 * gg + 16 * hh + ii).reshape(-1)
  out_cols = (32 * gg + 2 * ii + hh).reshape(-1)
  perm = jnp.zeros((DM, DM), jnp.float32).at[in_cols, out_cols].set(1.0)

  h1, als1, ald1 = _tc1(x, W1, as1, ad1, perm)
  u1, s1 = _sc_edge(src2d, dst2d, h1, als1, ald1, z128, z16)
  h2, als2, ald2 = _tc2(u1, s1, expand, b1.reshape(1, DM), W2, as2, ad2, perm)
  u2, s2 = _sc_edge(src2d, dst2d, h2, als2, ald2, z128, z16)
  out8 = _tc3(u2, s2, expand, red, b2.reshape(1, 16), _wp_pad(Wp),
              jnp.broadcast_to(bp.reshape(1, 1), (1, 8)))
  return out8[:, :1]


# speedup vs baseline: 1.1247x; 1.1247x over previous
import jax
import jax.numpy as jnp
from jax import lax
from jax.experimental import pallas as pl
from jax.experimental.pallas import tpu as pltpu
from jax.experimental.pallas import tpu_sc as plsc

N = 10000
E = 320000
FEAT = 128
HEADS = 8
HID = 16
DM = HEADS * HID

NC = 2
NS = 16
NW = NC * NS

E_TOT = E + N
EPB = 64
BLOCKS_PER_W = -(-E_TOT // (NW * EPB))
B_C0 = 180
B_C1 = 144
E_PAD = NW * EPB * BLOCKS_PER_W
ROWS_PER_SUB = N // NS

_EPS = 1e-16


def _tc1_body(x_ref, w_ref, as_ref, ad_ref, perm_ref, h_ref, als_ref,
              ald_ref):
  h = jnp.dot(x_ref[...], w_ref[...], preferred_element_type=jnp.float32)
  h_ref[...] = jnp.dot(h, perm_ref[...],
                       preferred_element_type=jnp.float32).astype(jnp.bfloat16)
  als_ref[...] = jnp.dot(h, as_ref[...], preferred_element_type=jnp.float32)
  ald_ref[...] = jnp.dot(h, ad_ref[...], preferred_element_type=jnp.float32)


def _tc1(x, w, a_s, a_d, perm):
  grid = 10
  blk = N // grid
  return pl.pallas_call(
      _tc1_body,
      grid=(grid,),
      in_specs=[
          pl.BlockSpec((blk, FEAT), lambda i: (i, 0)),
          pl.BlockSpec((FEAT, DM), lambda i: (0, 0)),
          pl.BlockSpec((DM, 16), lambda i: (0, 0)),
          pl.BlockSpec((DM, 16), lambda i: (0, 0)),
          pl.BlockSpec((DM, DM), lambda i: (0, 0)),
      ],
      out_specs=[
          pl.BlockSpec((blk, DM), lambda i: (i, 0)),
          pl.BlockSpec((blk, 16), lambda i: (i, 0)),
          pl.BlockSpec((blk, 16), lambda i: (i, 0)),
      ],
      out_shape=[
          jax.ShapeDtypeStruct((N, DM), jnp.bfloat16),
          jax.ShapeDtypeStruct((N, 16), jnp.float32),
          jax.ShapeDtypeStruct((N, 16), jnp.float32),
      ],
  )(x, w, a_s, a_d, perm)


def _tc2_body(u_ref, s_ref, exp_ref, b_ref, w_ref, as_ref, ad_ref, perm_ref,
              h2_ref, als_ref, ald_ref):
  u = u_ref[0] + u_ref[1]
  st = s_ref[0] + s_ref[1]
  r = 1.0 / (st + _EPS)
  rex = jnp.dot(r, exp_ref[...], preferred_element_type=jnp.float32)
  h1 = jnp.maximum(u * rex + b_ref[...], 0.0)
  h2 = jnp.dot(h1, w_ref[...], preferred_element_type=jnp.float32)
  h2_ref[...] = jnp.dot(h2, perm_ref[...],
                        preferred_element_type=jnp.float32).astype(jnp.bfloat16)
  als_ref[...] = jnp.dot(h2, as_ref[...], preferred_element_type=jnp.float32)
  ald_ref[...] = jnp.dot(h2, ad_ref[...], preferred_element_type=jnp.float32)


def _tc2(u, s, expand, b1, w2, a_s, a_d, perm):
  grid = 10
  blk = N // grid
  return pl.pallas_call(
      _tc2_body,
      grid=(grid,),
      in_specs=[
          pl.BlockSpec((2, blk, DM), lambda i: (0, i, 0)),
          pl.BlockSpec((2, blk, 16), lambda i: (0, i, 0)),
          pl.BlockSpec((16, DM), lambda i: (0, 0)),
          pl.BlockSpec((1, DM), lambda i: (0, 0)),
          pl.BlockSpec((DM, DM), lambda i: (0, 0)),
          pl.BlockSpec((DM, 16), lambda i: (0, 0)),
          pl.BlockSpec((DM, 16), lambda i: (0, 0)),
          pl.BlockSpec((DM, DM), lambda i: (0, 0)),
      ],
      out_specs=[
          pl.BlockSpec((blk, DM), lambda i: (i, 0)),
          pl.BlockSpec((blk, 16), lambda i: (i, 0)),
          pl.BlockSpec((blk, 16), lambda i: (i, 0)),
      ],
      out_shape=[
          jax.ShapeDtypeStruct((N, DM), jnp.bfloat16),
          jax.ShapeDtypeStruct((N, 16), jnp.float32),
          jax.ShapeDtypeStruct((N, 16), jnp.float32),
      ],
  )(u, s, expand, b1, w2, a_s, a_d, perm)


def _tc3_body(u_ref, s_ref, exp_ref, red_ref, b_ref, wp_ref, bp_ref, o_ref):
  u = u_ref[0] + u_ref[1]
  st = s_ref[0] + s_ref[1]
  r = 1.0 / (st + _EPS)
  rex = jnp.dot(r, exp_ref[...], preferred_element_type=jnp.float32)
  o = jnp.dot(u * rex, red_ref[...], preferred_element_type=jnp.float32)
  z = jnp.dot(o + b_ref[...], wp_ref[...], preferred_element_type=jnp.float32)
  o_ref[...] = jax.nn.sigmoid(z + bp_ref[...])


def _tc3(u, s, expand, red, b2, wp, bp):
  grid = 10
  blk = N // grid
  return pl.pallas_call(
      _tc3_body,
      grid=(grid,),
      in_specs=[
          pl.BlockSpec((2, blk, DM), lambda i: (0, i, 0)),
          pl.BlockSpec((2, blk, 16), lambda i: (0, i, 0)),
          pl.BlockSpec((16, DM), lambda i: (0, 0)),
          pl.BlockSpec((DM, 16), lambda i: (0, 0)),
          pl.BlockSpec((1, 16), lambda i: (0, 0)),
          pl.BlockSpec((16, 8), lambda i: (0, 0)),
          pl.BlockSpec((1, 8), lambda i: (0, 0)),
      ],
      out_specs=pl.BlockSpec((blk, 8), lambda i: (i, 0)),
      out_shape=jax.ShapeDtypeStruct((N, 8), jnp.float32),
  )(u, s, expand, red, b2, wp, bp)


def _sc_edge_body(src_hbm, dst_hbm, h_hbm, als_hbm, ald_hbm, z128_hbm,
                  z16_hbm, u_out, s_out,
                  src_v, dst_v, h_buf, out_buf, als_buf, ald_buf, p_buf,
                  u_shared, s_shared, sem_g0, sem_g1, sem_g2,
                  sem_i0, sem_i1, sem_i2, sem_i3, sem_i4, sem_i5, sem_sc):
  c = lax.axis_index("c")
  s = lax.axis_index("s")
  wid = s * NC + c

  nbase = s * ROWS_PER_SUB
  pltpu.sync_copy(z128_hbm, u_shared.at[pl.ds(nbase, ROWS_PER_SUB)])
  pltpu.sync_copy(z16_hbm, s_shared.at[pl.ds(nbase, ROWS_PER_SUB)])

  row0 = s * (B_C0 + B_C1) + c * B_C0
  nblocks = jnp.where(c == 0, B_C0, B_C1)
  sem_g = (sem_g0, sem_g1, sem_g2)
  sem_i = (sem_i0, sem_i1, sem_i2, sem_i3, sem_i4, sem_i5)

  def idx_copies(j, islot):
    return (
        pltpu.make_async_copy(src_hbm.at[row0 + j], src_v.at[islot],
                              sem_i[islot]),
        pltpu.make_async_copy(dst_hbm.at[row0 + j], dst_v.at[islot],
                              sem_i[islot]),
    )

  def gathers(hslot, islot):
    return (
        pltpu.make_async_copy(h_hbm.at[src_v.at[islot]], h_buf.at[hslot],
                              sem_g[hslot]),
        pltpu.make_async_copy(als_hbm.at[src_v.at[islot]], als_buf.at[hslot],
                              sem_g[hslot]),
        pltpu.make_async_copy(ald_hbm.at[dst_v.at[islot]], ald_buf.at[hslot],
                              sem_g[hslot]),
    )

  def scatters(oslot, hslot, islot):
    return (
        pltpu.make_async_copy(out_buf.at[oslot], u_shared.at[dst_v.at[islot]],
                              sem_sc),
        pltpu.make_async_copy(p_buf.at[hslot], s_shared.at[dst_v.at[islot]],
                              sem_sc),
    )

  def start(cps, add=False):
    for cp in cps:
      cp.start(add=add)

  def wait(cps):
    for cp in cps:
      cp.wait()

  def compute(j, hslot, oslot):
    gid0 = (row0 + j) * EPB

    @plsc.parallel_loop(0, EPB, unroll=4)
    def _(e):
      a = als_buf[hslot, e, :]
      d = ald_buf[hslot, e, :]
      t = a + d
      t = jnp.maximum(t, 0.2 * t)
      p = jnp.exp(t)
      valid = ((gid0 + e) < E_TOT).astype(jnp.float32)
      p = p * valid
      p_buf[hslot, e, :] = p
      for g4 in range(4):
        v = h_buf[hslot, e, pl.ds(g4 * 32, 32)]
        ha, hb = plsc.unpack(v, format=plsc.PackFormat.INTERLEAVED)
        out_buf[oslot, e, pl.ds(g4 * 32, HID)] = ha * p[2 * g4]
        out_buf[oslot, e, pl.ds(g4 * 32 + HID, HID)] = hb * p[2 * g4 + 1]

  for jj in range(4):
    start(idx_copies(jj, jj))
  wait(idx_copies(0, 0))
  wait(idx_copies(1, 1))
  start(gathers(0, 0))
  start(gathers(1, 1))

  def six_blocks(g, _):
    j0 = 6 * g
    for k in range(6):
      j = j0 + k
      hs = k % 3

      wait(gathers(hs, k))

      @pl.when(j >= 1)
      def _():
        wait(scatters((k + 1) % 2, (k + 2) % 3, (k + 5) % 6))

      compute(j, hs, k % 2)
      start(scatters(k % 2, hs, k), add=True)

      @pl.when(j + 2 < nblocks)
      def _():
        wait(idx_copies(j + 2, (k + 2) % 6))
        start(gathers((k + 2) % 3, (k + 2) % 6))

      @pl.when(j + 4 < nblocks)
      def _():
        start(idx_copies(j + 4, (k + 4) % 6))

    return 0

  lax.fori_loop(0, nblocks // 6, six_blocks, 0)
  wait(scatters(1, 2, 5))

  plsc.subcore_barrier()

  pltpu.sync_copy(u_shared.at[pl.ds(nbase, ROWS_PER_SUB)],
                  u_out.at[c, pl.ds(nbase, ROWS_PER_SUB)])
  pltpu.sync_copy(s_shared.at[pl.ds(nbase, ROWS_PER_SUB)],
                  s_out.at[c, pl.ds(nbase, ROWS_PER_SUB)])


def _sc_edge(src2d, dst2d, h, als, ald, z128, z16):
  mesh = plsc.VectorSubcoreMesh(core_axis_name="c", subcore_axis_name="s")
  fn = pl.kernel(
      _sc_edge_body,
      out_type=[
          jax.ShapeDtypeStruct((NC, N, DM), jnp.float32),
          jax.ShapeDtypeStruct((NC, N, 16), jnp.float32),
      ],
      mesh=mesh,
      compiler_params=pltpu.CompilerParams(use_tc_tiling_on_sc=False,
                                           needs_layout_passes=False),
      scratch_types=[
          pltpu.VMEM((6, EPB), jnp.int32),
          pltpu.VMEM((6, EPB), jnp.int32),
          pltpu.VMEM((3, EPB, DM), jnp.bfloat16),
          pltpu.VMEM((2, EPB, DM), jnp.float32),
          pltpu.VMEM((3, EPB, 16), jnp.float32),
          pltpu.VMEM((3, EPB, 16), jnp.float32),
          pltpu.VMEM((3, EPB, 16), jnp.float32),
          pltpu.VMEM_SHARED((N, DM), jnp.float32),
          pltpu.VMEM_SHARED((N, 16), jnp.float32),
      ] + [pltpu.SemaphoreType.DMA] * 10,
  )
  return fn(src2d, dst2d, h, als, ald, z128, z16)


def _head_proj(a):
  flat = a.reshape(DM)
  rows = jnp.arange(DM)
  return jnp.zeros((DM, 16), jnp.float32).at[rows, rows // HID].set(flat)


def _wp_pad(wp):
  return jnp.concatenate([wp, jnp.zeros((HID, 7), jnp.float32)], axis=1)


@jax.jit
def kernel(edge_index, x, W1, a_src1, a_dst1, b1, W2, a_src2, a_dst2, b2,
           Wp, bp):
  loops = jnp.arange(N, dtype=jnp.int32)
  pad = jnp.zeros((E_PAD - E_TOT,), jnp.int32)
  src2d = jnp.concatenate(
      [edge_index[0].astype(jnp.int32), loops, pad]).reshape(-1, EPB)
  dst2d = jnp.concatenate(
      [edge_index[1].astype(jnp.int32), loops, pad]).reshape(-1, EPB)

  as1 = _head_proj(a_src1)
  ad1 = _head_proj(a_dst1)
  as2 = _head_proj(a_src2)
  ad2 = _head_proj(a_dst2)

  expand = (jnp.arange(DM)[None, :] // HID ==
            jnp.arange(16)[:, None]).astype(jnp.float32)
  red = ((jnp.arange(DM)[:, None] % HID ==
          jnp.arange(16)[None, :]).astype(jnp.float32) / HEADS)

  z128 = jnp.zeros((ROWS_PER_SUB, DM), jnp.float32)
  z16 = jnp.zeros((ROWS_PER_SUB, 16), jnp.float32)

  gg = jnp.arange(4)[:, None, None]
  hh = jnp.arange(2)[None, :, None]
  ii = jnp.arange(16)[None, None, :]
  in_cols = (32 * gg + 16 * hh + ii).reshape(-1)
  out_cols = (32 * gg + 2 * ii + hh).reshape(-1)
  perm = jnp.zeros((DM, DM), jnp.float32).at[in_cols, out_cols].set(1.0)

  h1, als1, ald1 = _tc1(x, W1, as1, ad1, perm)
  u1, s1 = _sc_edge(src2d, dst2d, h1, als1, ald1, z128, z16)
  h2, als2, ald2 = _tc2(u1, s1, expand, b1.reshape(1, DM), W2, as2, ad2, perm)
  u2, s2 = _sc_edge(src2d, dst2d, h2, als2, ald2, z128, z16)
  out8 = _tc3(u2, s2, expand, red, b2.reshape(1, 16), _wp_pad(Wp),
              jnp.broadcast_to(bp.reshape(1, 1), (1, 8)))
  return out8[:, :1]
